# baseline (speedup 1.0000x reference)
import functools

import jax
import jax.numpy as jnp
import numpy as np
from jax import lax
from jax.experimental import pallas as pl
from jax.experimental.pallas import tpu as pltpu
from jax.experimental.pallas import tpu_sc as plsc

KERNEL = 15
PAD = 10
RES = 128
PIMG = RES + 2 * PAD
L = RES + 2 * PAD - KERNEL + 1
NKEYS = 16384
DPAD = 256

QT = 2048
KT = 512
NQ = 11
NK = NKEYS // KT
NQPAD = NQ * QT

NW = 32
BPW = NQPAD // NW
CH = 64
NCH = BPW // CH

ACC_R = 176
ACC_C = 160


def _argmin_body(qt_ref, k_ref, out_ref, best_ref, bidx_ref, qq_ref):
    ki = pl.program_id(1)
    qt = qt_ref[...]
    kt = k_ref[...]
    s = lax.dot_general(qt, kt, (((0,), (1,)), ((), ())),
                        preferred_element_type=jnp.float32)

    @pl.when(ki == 0)
    def _():
        qq_ref[...] = jnp.sum(qt * qt, axis=0)[:, None]

    ksq = jnp.sum(kt * kt, axis=1)
    d = (qq_ref[...] - 2.0 * s) + ksq[None, :]
    m = jnp.min(d, axis=1, keepdims=True)
    jj = lax.broadcasted_iota(jnp.int32, (QT, KT), 1)
    cand = jnp.where(d == m, jj, jnp.int32(2**30))
    ji = jnp.min(cand, axis=1, keepdims=True) + ki * KT

    @pl.when(ki == 0)
    def _():
        best_ref[...] = jnp.full((QT, 1), jnp.inf, jnp.float32)
        bidx_ref[...] = jnp.zeros((QT, 1), jnp.int32)

    better = m < best_ref[...]
    best_ref[...] = jnp.where(better, m, best_ref[...])
    bidx_ref[...] = jnp.where(better, ji, bidx_ref[...])

    @pl.when(ki == NK - 1)
    def _():
        out_ref[...] = bidx_ref[...].reshape(1, 1, QT)


_argmin_call = pl.pallas_call(
    _argmin_body,
    grid=(NQ, NK),
    in_specs=[
        pl.BlockSpec((DPAD, QT), lambda qi, ki: (0, qi)),
        pl.BlockSpec((KT, DPAD), lambda qi, ki: (ki, 0)),
    ],
    out_specs=pl.BlockSpec((1, 1, QT), lambda qi, ki: (qi, 0, 0)),
    out_shape=jax.ShapeDtypeStruct((NQ, 1, QT), jnp.int32),
    scratch_shapes=[
        pltpu.VMEM((QT, 1), jnp.float32),
        pltpu.VMEM((QT, 1), jnp.int32),
        pltpu.VMEM((QT, 1), jnp.float32),
    ],
    compiler_params=pltpu.CompilerParams(
        dimension_semantics=("parallel", "arbitrary"),
        fuse_transposed_lhs_in_matmul=True),
)


def _sc_gather_fold_body(keys_hbm, idx_hbm, out_hbm, idx_v, rows_v, acc_v, sem):
    c = lax.axis_index("c")
    s = lax.axis_index("s")
    wid = s * 2 + c
    base = wid * BPW
    pltpu.sync_copy(idx_hbm.at[pl.ds(base, BPW)], idx_v)

    zero16 = jnp.zeros((16,), jnp.float32)

    @plsc.parallel_loop(0, ACC_R, unroll=4)
    def _zrow(r):
        for t in range(ACC_C // 16):
            acc_v[r, pl.ds(16 * t, 16)] = zero16

    lane = lax.iota(jnp.int32, 16)
    st_r = []
    st_c = []
    for t in range(16):
        k = lane + 16 * t
        i = k // 15
        j = k - i * 15
        valid = k < 225
        st_r.append(jnp.where(valid, i, 0))
        st_c.append(jnp.where(valid, j, 0))

    r0 = base // PIMG
    c0 = base - r0 * PIMG

    def chunk_body(ch, rc):
        r0c, c0c = rc
        copy = pltpu.async_copy(
            keys_hbm.at[idx_v.at[pl.ds(ch * CH, CH)]], rows_v, sem)
        copy.wait()

        @plsc.parallel_loop(0, CH, unroll=4)
        def _patch(l):
            l2 = (l & 3) * 16 + (l >> 2)
            c1 = c0c + l2
            ovf = (c1 >= PIMG).astype(jnp.int32)
            r = r0c + ovf
            cc = c1 - PIMG * ovf
            pvalid = jnp.logical_and(r < L, cc < L)
            rr = jnp.where(pvalid, r, 148)
            cb = jnp.where(pvalid, cc, 0)
            for t in range(16):
                vals = rows_v[l2, pl.ds(16 * t, 16)]
                plsc.addupdate_scatter(
                    acc_v, [st_r[t] + rr, st_c[t] + cb], vals)

        c2 = c0c + CH
        ovf2 = (c2 >= PIMG).astype(jnp.int32)
        return (r0c + ovf2, c2 - PIMG * ovf2)

    lax.fori_loop(0, NCH, chunk_body, (r0, c0))
    pltpu.sync_copy(acc_v, out_hbm.at[wid])


@functools.cache
def _sc_gather_fold():
    return pl.kernel(
        _sc_gather_fold_body,
        out_type=jax.ShapeDtypeStruct((NW, ACC_R, ACC_C), jnp.float32),
        mesh=plsc.VectorSubcoreMesh(core_axis_name="c", subcore_axis_name="s"),
        compiler_params=pltpu.CompilerParams(needs_layout_passes=False),
        scratch_types=[
            pltpu.VMEM((BPW,), jnp.int32),
            pltpu.VMEM((CH, DPAD), jnp.float32),
            pltpu.VMEM((ACC_R, ACC_C), jnp.float32),
            pltpu.SemaphoreType.DMA,
        ],
    )


def _epilogue_body(acc_ref, s_ref, out_ref):
    f = jnp.sum(acc_ref[...], axis=0)
    f = f[PAD:PAD + RES, PAD:PAD + RES]
    sel = s_ref[...]
    r1 = lax.dot_general(sel, f, (((1,), (0,)), ((), ())),
                         precision=lax.Precision.HIGHEST,
                         preferred_element_type=jnp.float32)
    r2 = lax.dot_general(r1, sel, (((1,), (1,)), ((), ())),
                         precision=lax.Precision.HIGHEST,
                         preferred_element_type=jnp.float32)
    out_ref[...] = r2 / jnp.max(r2)


_epilogue_call = pl.pallas_call(
    _epilogue_body,
    in_specs=[
        pl.BlockSpec((NW, ACC_R, ACC_C), lambda: (0, 0, 0)),
        pl.BlockSpec((64, RES), lambda: (0, 0)),
    ],
    out_specs=pl.BlockSpec((64, 64), lambda: (0, 0)),
    out_shape=jax.ShapeDtypeStruct((64, 64), jnp.float32),
)

_SEL = np.zeros((64, RES), np.float32)
_SEL[np.arange(64), np.floor((np.arange(64) + 0.5) * 2.0).astype(np.int64)] = 1.0


def kernel(image, keys):
    img = jax.image.resize(image, (RES, RES), method='nearest')
    flat = jnp.pad(img, PAD).reshape(-1)
    flat = jnp.pad(flat, (0, NQPAD + (KERNEL - 1) * (PIMG + 1) - flat.shape[0]))
    qt = jnp.stack(
        [lax.dynamic_slice(flat, (i * PIMG + j,), (NQPAD,))
         for i in range(KERNEL) for j in range(KERNEL)], axis=0)
    qt = jnp.pad(qt, ((0, DPAD - KERNEL * KERNEL), (0, 0)))
    keys_p = jnp.pad(keys, ((0, 0), (0, DPAD - KERNEL * KERNEL)))

    idx = _argmin_call(qt, keys_p).reshape(-1)
    acc = _sc_gather_fold()(keys_p, idx)
    return _epilogue_call(acc, jnp.asarray(_SEL))


# speedup vs baseline: 1.2452x; 1.2452x over previous
import functools

import jax
import jax.numpy as jnp
import numpy as np
from jax import lax
from jax.experimental import pallas as pl
from jax.experimental.pallas import tpu as pltpu
from jax.experimental.pallas import tpu_sc as plsc

KERNEL = 15
PAD = 10
RES = 128
PIMG = RES + 2 * PAD
L = RES + 2 * PAD - KERNEL + 1
NKEYS = 16384
DPAD = 256

QT = 2048
KT = 512
NQ = 11
NK = NKEYS // KT
NQPAD = NQ * QT

NW = 32
BPW = NQPAD // NW
CH = 64
NCH = BPW // CH

ACC_R = 176
ACC_C = 160


def _argmin_body(qt_ref, k_ref, out_ref, best_ref, bidx_ref):
    ki = pl.program_id(1)
    qt = qt_ref[...]
    kt = k_ref[...]
    s = lax.dot_general(qt, kt, (((0,), (1,)), ((), ())),
                        preferred_element_type=jnp.float32)
    qq = jnp.sum(qt * qt, axis=0)[:, None]
    ksq = jnp.sum(kt * kt, axis=1)
    d = (qq - 2.0 * s) + ksq[None, :]
    m = jnp.min(d, axis=1, keepdims=True)
    jj = lax.broadcasted_iota(jnp.int32, (QT, KT), 1)
    cand = jnp.where(d == m, jj, jnp.int32(2**30))
    ji = jnp.min(cand, axis=1, keepdims=True) + ki * KT

    @pl.when(ki == 0)
    def _():
        best_ref[...] = jnp.full((QT, 1), jnp.inf, jnp.float32)
        bidx_ref[...] = jnp.zeros((QT, 1), jnp.int32)

    better = m < best_ref[...]
    best_ref[...] = jnp.where(better, m, best_ref[...])
    bidx_ref[...] = jnp.where(better, ji, bidx_ref[...])

    @pl.when(ki == NK - 1)
    def _():
        out_ref[...] = bidx_ref[...].reshape(1, 1, QT)


_argmin_call = pl.pallas_call(
    _argmin_body,
    grid=(NQ, NK),
    in_specs=[
        pl.BlockSpec((DPAD, QT), lambda qi, ki: (0, qi)),
        pl.BlockSpec((KT, DPAD), lambda qi, ki: (ki, 0)),
    ],
    out_specs=pl.BlockSpec((1, 1, QT), lambda qi, ki: (qi, 0, 0)),
    out_shape=jax.ShapeDtypeStruct((NQ, 1, QT), jnp.int32),
    scratch_shapes=[
        pltpu.VMEM((QT, 1), jnp.float32),
        pltpu.VMEM((QT, 1), jnp.int32),
    ],
    compiler_params=pltpu.CompilerParams(
        dimension_semantics=("parallel", "arbitrary"),
        fuse_transposed_lhs_in_matmul=True),
)


def _sc_gather_fold_body(keys_hbm, idx_hbm, out_hbm, idx_v, rows_v, acc_v, sem):
    c = lax.axis_index("c")
    s = lax.axis_index("s")
    wid = s * 2 + c
    base = wid * BPW
    pltpu.sync_copy(idx_hbm.at[pl.ds(base, BPW)], idx_v)

    zero16 = jnp.zeros((16,), jnp.float32)

    @plsc.parallel_loop(0, ACC_R, unroll=4)
    def _zrow(r):
        for t in range(ACC_C // 16):
            acc_v[r, pl.ds(16 * t, 16)] = zero16

    lane = lax.iota(jnp.int32, 16)
    st_r = []
    st_c = []
    for t in range(16):
        k = lane + 16 * t
        i = k // 15
        j = k - i * 15
        valid = k < 225
        st_r.append(jnp.where(valid, i, 0))
        st_c.append(jnp.where(valid, j, 0))

    r0 = base // PIMG
    c0 = base - r0 * PIMG

    def chunk_body(ch, rc):
        r0c, c0c = rc
        copy = pltpu.async_copy(
            keys_hbm.at[idx_v.at[pl.ds(ch * CH, CH)]], rows_v, sem)
        copy.wait()

        @plsc.parallel_loop(0, CH, unroll=4)
        def _patch(l):
            l2 = (l & 3) * 16 + (l >> 2)
            c1 = c0c + l2
            ovf = (c1 >= PIMG).astype(jnp.int32)
            r = r0c + ovf
            cc = c1 - PIMG * ovf
            pvalid = jnp.logical_and(r < L, cc < L)
            rr = jnp.where(pvalid, r, 148)
            cb = jnp.where(pvalid, cc, 0)
            for t in range(16):
                vals = rows_v[l2, pl.ds(16 * t, 16)]
                plsc.addupdate_scatter(
                    acc_v, [st_r[t] + rr, st_c[t] + cb], vals)

        c2 = c0c + CH
        ovf2 = (c2 >= PIMG).astype(jnp.int32)
        return (r0c + ovf2, c2 - PIMG * ovf2)

    lax.fori_loop(0, NCH, chunk_body, (r0, c0))
    pltpu.sync_copy(acc_v, out_hbm.at[wid])


@functools.cache
def _sc_gather_fold():
    return pl.kernel(
        _sc_gather_fold_body,
        out_type=jax.ShapeDtypeStruct((NW, ACC_R, ACC_C), jnp.float32),
        mesh=plsc.VectorSubcoreMesh(core_axis_name="c", subcore_axis_name="s"),
        compiler_params=pltpu.CompilerParams(needs_layout_passes=False),
        scratch_types=[
            pltpu.VMEM((BPW,), jnp.int32),
            pltpu.VMEM((CH, DPAD), jnp.float32),
            pltpu.VMEM((ACC_R, ACC_C), jnp.float32),
            pltpu.SemaphoreType.DMA,
        ],
    )


def _epilogue_body(acc_ref, s_ref, out_ref):
    f = jnp.sum(acc_ref[...], axis=0)
    f = f[PAD:PAD + RES, PAD:PAD + RES]
    sel = s_ref[...]
    r1 = lax.dot_general(sel, f, (((1,), (0,)), ((), ())),
                         precision=lax.Precision.HIGHEST,
                         preferred_element_type=jnp.float32)
    r2 = lax.dot_general(r1, sel, (((1,), (1,)), ((), ())),
                         precision=lax.Precision.HIGHEST,
                         preferred_element_type=jnp.float32)
    out_ref[...] = r2 / jnp.max(r2)


_epilogue_call = pl.pallas_call(
    _epilogue_body,
    in_specs=[
        pl.BlockSpec((NW, ACC_R, ACC_C), lambda: (0, 0, 0)),
        pl.BlockSpec((64, RES), lambda: (0, 0)),
    ],
    out_specs=pl.BlockSpec((64, 64), lambda: (0, 0)),
    out_shape=jax.ShapeDtypeStruct((64, 64), jnp.float32),
)

_SEL = np.zeros((64, RES), np.float32)
_SEL[np.arange(64), np.floor((np.arange(64) + 0.5) * 2.0).astype(np.int64)] = 1.0


def kernel(image, keys):
    img = jax.image.resize(image, (RES, RES), method='nearest')
    flat = jnp.pad(img, PAD).reshape(-1)
    flat = jnp.pad(flat, (0, NQPAD + (KERNEL - 1) * (PIMG + 1) - flat.shape[0]))
    qt = jnp.stack(
        [lax.dynamic_slice(flat, (i * PIMG + j,), (NQPAD,))
         for i in range(KERNEL) for j in range(KERNEL)], axis=0)
    qt = jnp.pad(qt, ((0, DPAD - KERNEL * KERNEL), (0, 0)))
    keys_p = jnp.pad(keys, ((0, 0), (0, DPAD - KERNEL * KERNEL)))

    idx = _argmin_call(qt, keys_p).reshape(-1)
    acc = _sc_gather_fold()(keys_p, idx)
    return _epilogue_call(acc, jnp.asarray(_SEL))


# speedup vs baseline: 1.3555x; 1.0886x over previous
import functools

import jax
import jax.numpy as jnp
import numpy as np
from jax import lax
from jax.experimental import pallas as pl
from jax.experimental.pallas import tpu as pltpu
from jax.experimental.pallas import tpu_sc as plsc

KERNEL = 15
PAD = 10
RES = 128
PIMG = RES + 2 * PAD
L = RES + 2 * PAD - KERNEL + 1
NKEYS = 16384
DPAD = 256

QT = 2048
KT = 512
NQ = 11
NK = NKEYS // KT
NQPAD = NQ * QT

NW = 32
BPW = NQPAD // NW
CH = 64
NCH = BPW // CH

ACC_R = 176
ACC_C = 160


def _argmin_body(qt_ref, k_ref, out_ref, best_ref, bidx_ref):
    ki = pl.program_id(1)
    qt = qt_ref[...]
    kt = k_ref[...]
    v = lax.dot_general(qt, kt, (((0,), (1,)), ((), ())),
                        preferred_element_type=jnp.float32)
    m = jnp.max(v, axis=1, keepdims=True)
    jj = lax.broadcasted_iota(jnp.int32, (QT, KT), 1)
    cand = jnp.where(v == m, jj, jnp.int32(2**30))
    ji = jnp.min(cand, axis=1, keepdims=True) + ki * KT

    @pl.when(ki == 0)
    def _():
        best_ref[...] = jnp.full((QT, 1), -jnp.inf, jnp.float32)
        bidx_ref[...] = jnp.zeros((QT, 1), jnp.int32)

    better = m > best_ref[...]
    best_ref[...] = jnp.where(better, m, best_ref[...])
    bidx_ref[...] = jnp.where(better, ji, bidx_ref[...])

    @pl.when(ki == NK - 1)
    def _():
        out_ref[...] = bidx_ref[...].reshape(1, 1, QT)


_argmin_call = pl.pallas_call(
    _argmin_body,
    grid=(NQ, NK),
    in_specs=[
        pl.BlockSpec((DPAD, QT), lambda qi, ki: (0, qi)),
        pl.BlockSpec((KT, DPAD), lambda qi, ki: (ki, 0)),
    ],
    out_specs=pl.BlockSpec((1, 1, QT), lambda qi, ki: (qi, 0, 0)),
    out_shape=jax.ShapeDtypeStruct((NQ, 1, QT), jnp.int32),
    scratch_shapes=[
        pltpu.VMEM((QT, 1), jnp.float32),
        pltpu.VMEM((QT, 1), jnp.int32),
    ],
    compiler_params=pltpu.CompilerParams(
        dimension_semantics=("parallel", "arbitrary"),
        fuse_transposed_lhs_in_matmul=True),
)


def _sc_gather_fold_body(keys_hbm, idx_hbm, out_hbm, idx_v, rows_v, acc_v, sem):
    c = lax.axis_index("c")
    s = lax.axis_index("s")
    wid = s * 2 + c
    base = wid * BPW
    pltpu.sync_copy(idx_hbm.at[pl.ds(base, BPW)], idx_v)

    zero16 = jnp.zeros((16,), jnp.float32)

    @plsc.parallel_loop(0, ACC_R, unroll=4)
    def _zrow(r):
        for t in range(ACC_C // 16):
            acc_v[r, pl.ds(16 * t, 16)] = zero16

    lane = lax.iota(jnp.int32, 16)
    st_r = []
    st_c = []
    for t in range(16):
        k = lane + 16 * t
        i = k // 15
        j = k - i * 15
        valid = k < 225
        st_r.append(jnp.where(valid, i, 0))
        st_c.append(jnp.where(valid, j, 0))

    r0 = base // PIMG
    c0 = base - r0 * PIMG

    def chunk_body(ch, rc):
        r0c, c0c = rc
        copy = pltpu.async_copy(
            keys_hbm.at[idx_v.at[pl.ds(ch * CH, CH)]], rows_v, sem)
        copy.wait()

        @plsc.parallel_loop(0, CH, unroll=4)
        def _patch(l):
            l2 = (l & 3) * 16 + (l >> 2)
            c1 = c0c + l2
            ovf = (c1 >= PIMG).astype(jnp.int32)
            r = r0c + ovf
            cc = c1 - PIMG * ovf
            pvalid = jnp.logical_and(r < L, cc < L)
            rr = jnp.where(pvalid, r, 148)
            cb = jnp.where(pvalid, cc, 0)
            for t in range(16):
                vals = rows_v[l2, pl.ds(16 * t, 16)]
                plsc.addupdate_scatter(
                    acc_v, [st_r[t] + rr, st_c[t] + cb], vals)

        c2 = c0c + CH
        ovf2 = (c2 >= PIMG).astype(jnp.int32)
        return (r0c + ovf2, c2 - PIMG * ovf2)

    lax.fori_loop(0, NCH, chunk_body, (r0, c0))
    pltpu.sync_copy(acc_v, out_hbm.at[wid])


@functools.cache
def _sc_gather_fold():
    return pl.kernel(
        _sc_gather_fold_body,
        out_type=jax.ShapeDtypeStruct((NW, ACC_R, ACC_C), jnp.float32),
        mesh=plsc.VectorSubcoreMesh(core_axis_name="c", subcore_axis_name="s"),
        compiler_params=pltpu.CompilerParams(needs_layout_passes=False),
        scratch_types=[
            pltpu.VMEM((BPW,), jnp.int32),
            pltpu.VMEM((CH, DPAD), jnp.float32),
            pltpu.VMEM((ACC_R, ACC_C), jnp.float32),
            pltpu.SemaphoreType.DMA,
        ],
    )


def _epilogue_body(acc_ref, s_ref, out_ref):
    f = jnp.sum(acc_ref[...], axis=0)
    f = f[PAD:PAD + RES, PAD:PAD + RES]
    sel = s_ref[...]
    r1 = lax.dot_general(sel, f, (((1,), (0,)), ((), ())),
                         precision=lax.Precision.HIGHEST,
                         preferred_element_type=jnp.float32)
    r2 = lax.dot_general(r1, sel, (((1,), (1,)), ((), ())),
                         precision=lax.Precision.HIGHEST,
                         preferred_element_type=jnp.float32)
    out_ref[...] = r2 / jnp.max(r2)


_epilogue_call = pl.pallas_call(
    _epilogue_body,
    in_specs=[
        pl.BlockSpec((NW, ACC_R, ACC_C), lambda: (0, 0, 0)),
        pl.BlockSpec((64, RES), lambda: (0, 0)),
    ],
    out_specs=pl.BlockSpec((64, 64), lambda: (0, 0)),
    out_shape=jax.ShapeDtypeStruct((64, 64), jnp.float32),
)

_SEL = np.zeros((64, RES), np.float32)
_SEL[np.arange(64), np.floor((np.arange(64) + 0.5) * 2.0).astype(np.int64)] = 1.0


def kernel(image, keys):
    img = jax.image.resize(image, (RES, RES), method='nearest')
    flat = jnp.pad(img, PAD).reshape(-1)
    flat = jnp.pad(flat, (0, NQPAD + (KERNEL - 1) * (PIMG + 1) - flat.shape[0]))
    ones = jnp.ones((NQPAD,), jnp.float32)
    qt = jnp.stack(
        [lax.dynamic_slice(flat, (i * PIMG + j,), (NQPAD,))
         for i in range(KERNEL) for j in range(KERNEL)]
        + [ones, ones], axis=0)
    qt = jnp.pad(qt, ((0, DPAD - KERNEL * KERNEL - 2), (0, 0)))
    keys_p = jnp.pad(keys, ((0, 0), (0, DPAD - KERNEL * KERNEL)))
    ksq_col = -0.5 * jnp.sum(keys * keys, axis=1)
    hi = ksq_col.astype(jnp.bfloat16).astype(jnp.float32)
    lo = ksq_col - hi
    keys_aug = keys_p.at[:, KERNEL * KERNEL].set(hi)
    keys_aug = keys_aug.at[:, KERNEL * KERNEL + 1].set(lo)

    idx = _argmin_call(qt, keys_aug).reshape(-1)
    acc = _sc_gather_fold()(keys_p, idx)
    return _epilogue_call(acc, jnp.asarray(_SEL))
